# baseline (speedup 1.0000x reference)
import functools

import jax
import jax.numpy as jnp
from jax import lax
from jax.experimental import pallas as pl
from jax.experimental.pallas import tpu as pltpu
from jax.experimental.pallas import tpu_sc as plsc

B, L, F = 4096, 50, 10
D = 32
NT = 8
G = 128
V = 16

NC, NS = 2, 16
NW = NC * NS

BL = B * L
NBLK = BL // G
STEPS = NBLK // NW
RAW = G * F

_MESH = plsc.VectorSubcoreMesh(core_axis_name="c", subcore_axis_name="s")


@functools.partial(
    pl.kernel,
    out_type=(
        jax.ShapeDtypeStruct((NBLK, G, D), jnp.float32),
        jax.ShapeDtypeStruct((NBLK * NT, G, D), jnp.float32),
    ),
    mesh=_MESH,
    compiler_params=pltpu.CompilerParams(
        use_tc_tiling_on_sc=False, needs_layout_passes=False),
    scratch_types=(
        pltpu.VMEM((2, RAW), jnp.int32),
        pltpu.VMEM((2, NT, G), jnp.int32),
        pltpu.VMEM((2, G), jnp.int32),
        pltpu.VMEM((2, NT, G, D), jnp.float32),
        pltpu.VMEM((2, G, D), jnp.float32),
        pltpu.SemaphoreType.DMA,
        pltpu.SemaphoreType.DMA,
    ),
)
def _sc_embed(x_hbm, rel_tab_hbm, type_tab_hbm, rel_out_hbm, type_out_hbm,
              raw_v, tidx_v, ridx_v, trows_v, rrows_v, sem0, sem1):
    wid = lax.axis_index("s") * NC + lax.axis_index("c")
    sems = (sem0, sem1)

    lane = lax.iota(jnp.int32, V)
    patt_t = (lane >> 3) * F + (lane & 7)
    patt_r = lane * F + (F - 1)

    def load_fire(slot, step):
        blk = wid + step * NW
        pltpu.sync_copy(x_hbm.at[pl.ds(blk * RAW, RAW)], raw_v.at[slot])
        raw = raw_v.at[slot]
        for j in range(NT):
            row = tidx_v.at[slot].at[j]
            for cc in range(0, G, V):
                off = patt_t + ((j * G + cc) // 8) * F
                row[pl.ds(cc, V)] = plsc.load_gather(raw, [off])
        rrow = ridx_v.at[slot]
        for cc in range(0, G, V):
            rrow[pl.ds(cc, V)] = plsc.load_gather(raw, [patt_r + cc * F])
        for j in range(NT):
            pltpu.async_copy(
                type_tab_hbm.at[tidx_v.at[slot].at[j]],
                trows_v.at[slot].at[j], sems[slot])
        pltpu.async_copy(
            rel_tab_hbm.at[ridx_v.at[slot]], rrows_v.at[slot], sems[slot])

    def drain(slot):
        for j in range(NT):
            pltpu.make_async_copy(
                type_tab_hbm.at[tidx_v.at[slot].at[j]],
                trows_v.at[slot].at[j], sems[slot]).wait()
        pltpu.make_async_copy(
            rel_tab_hbm.at[ridx_v.at[slot]], rrows_v.at[slot],
            sems[slot]).wait()

    def store(slot, step):
        blk = wid + step * NW
        pltpu.sync_copy(trows_v.at[slot],
                        type_out_hbm.at[pl.ds(blk * NT, NT)])
        pltpu.sync_copy(rrows_v.at[slot], rel_out_hbm.at[blk])

    n_pairs = STEPS // 2
    load_fire(0, 0)

    def pair(p, carry):
        s0 = 2 * p
        load_fire(1, s0 + 1)
        drain(0)
        store(0, s0)

        @pl.when(p < n_pairs - 1)
        def _():
            load_fire(0, s0 + 2)

        drain(1)
        store(1, s0 + 1)
        return carry

    lax.fori_loop(0, n_pairs, pair, 0)


def kernel(x, rel_table, type_table):
    rel_out, type_out = _sc_embed(x.reshape(-1), rel_table, type_table)
    return (rel_out.reshape(B, L, D), type_out.reshape(B, L, NT, D))
